# baseline (speedup 1.0000x reference)
import jax
import jax.numpy as jnp
from jax import lax
from jax.experimental import pallas as pl
from jax.experimental.pallas import tpu as pltpu
from jax.experimental.pallas import tpu_sc as plsc

_V = 1000000
_H = 16
_B = 16384

_NC = 2
_NS = 16
_NW = _NC * _NS
_BPW = _B // _NW
_RPW = 2 * _BPW
_KCH = 8
_CW = _RPW // _KCH
_NG = _BPW // 16

_CB = 262144
_GRID = -(-_V // _CB)
_DROWS = _GRID * _CB // 128


def _tc_project(embt_ref, dw_ref, d_ref):
    blk = embt_ref[...]
    dw = dw_ref[...]
    s = jnp.sum(blk * dw[:, 0:1], axis=0)
    d_ref[...] = s.reshape(_CB // 128, 128)


def _sc_body(idx_hbm, d16_hbm, g_hbm, out_hbm, idx_v, hi_v, rows_v, g_v, out_v, sem):
    wid = lax.axis_index("s") * _NC + lax.axis_index("c")

    pltpu.sync_copy(idx_hbm.at[wid], idx_v)
    pltpu.sync_copy(g_hbm, g_v)

    for k in range(_KCH):
        for t in range(_CW // 16):
            v = idx_v[k, pl.ds(t * 16, 16)]
            hi_v[k, pl.ds(t * 16, 16)] = lax.shift_right_logical(v, 4)

    copies = [
        pltpu.async_copy(
            d16_hbm.at[hi_v.at[k]], rows_v.at[pl.ds(k * _CW, _CW)], sem
        )
        for k in range(_KCH)
    ]
    for c in copies:
        c.wait()

    iot = lax.iota(jnp.int32, 16)
    zero16 = jnp.zeros((16,), jnp.int32)
    one16 = jnp.ones((16,), jnp.int32)
    w0v = g_v[0, :]
    w1v = g_v[1, :]
    cdv = g_v[2, :]

    def group(g, carry):
        r0 = g * 32 + iot * 2
        r1 = r0 + 1
        lo0 = plsc.load_gather(idx_v, [lax.shift_right_logical(r0, 7), r0 & 127]) & 15
        lo1 = plsc.load_gather(idx_v, [lax.shift_right_logical(r1, 7), r1 & 127]) & 15
        v0 = plsc.load_gather(rows_v, [r0, lo0])
        v1 = plsc.load_gather(rows_v, [r1, lo1])
        d = w0v * v0 + w1v * v1 + cdv
        e = jnp.exp(d)
        o0 = 1.0 / (1.0 + e)
        o1 = 1.0 - o0
        bi = g * 16 + iot
        plsc.store_scatter(out_v, [bi, zero16], o0)
        plsc.store_scatter(out_v, [bi, one16], o1)
        return carry

    lax.fori_loop(0, _NG, group, 0)

    pltpu.sync_copy(out_v, out_hbm.at[pl.ds(wid * _BPW, _BPW)])


def kernel(input, emb, W1, b1, W2, b2):
    idx = input.astype(jnp.int32).reshape(_NW, _KCH, _CW)
    dw = W2[1] - W2[0]
    cd = b1[0] * jnp.sum(dw) + (b2[1] - b2[0])
    gconst = jnp.stack(
        [
            jnp.full((16,), W1[0, 0], jnp.float32),
            jnp.full((16,), W1[0, 1], jnp.float32),
            jnp.full((16,), cd, jnp.float32),
        ]
    )

    d2 = pl.pallas_call(
        _tc_project,
        grid=(_GRID,),
        in_specs=[
            pl.BlockSpec((_H, _CB), lambda i: (0, i)),
            pl.BlockSpec((_H, 128), lambda i: (0, 0)),
        ],
        out_specs=pl.BlockSpec((_CB // 128, 128), lambda i: (i, 0)),
        out_shape=jax.ShapeDtypeStruct((_DROWS, 128), jnp.float32),
    )(emb.T, jnp.broadcast_to(dw[:, None], (_H, 128)))
    d16 = d2.reshape(_DROWS * 8, _H)

    mesh = plsc.VectorSubcoreMesh(
        core_axis_name="c", subcore_axis_name="s", num_cores=_NC, num_subcores=_NS
    )
    run = pl.kernel(
        _sc_body,
        out_type=jax.ShapeDtypeStruct((_B, 2), jnp.float32),
        mesh=mesh,
        compiler_params=pltpu.CompilerParams(
            needs_layout_passes=False, use_tc_tiling_on_sc=False
        ),
        scratch_types=[
            pltpu.VMEM((_KCH, _CW), jnp.int32),
            pltpu.VMEM((_KCH, _CW), jnp.int32),
            pltpu.VMEM((_RPW, _H), jnp.float32),
            pltpu.VMEM((3, 16), jnp.float32),
            pltpu.VMEM((_BPW, 2), jnp.float32),
            pltpu.SemaphoreType.DMA,
        ],
    )
    return run(idx, d16, gconst)


# speedup vs baseline: 1.1190x; 1.1190x over previous
import jax
import jax.numpy as jnp
from jax import lax
from jax.experimental import pallas as pl
from jax.experimental.pallas import tpu as pltpu
from jax.experimental.pallas import tpu_sc as plsc

_V = 1000000
_H = 16
_B = 16384

_NC = 2
_NS = 16
_NW = _NC * _NS
_BPW = _B // _NW
_RPW = 2 * _BPW
_KCH = 8
_CW = _RPW // _KCH
_NG = _BPW // 16

_CB = 131072
_GRID = -(-_V // _CB)
_DROWS = _GRID * _CB // 128
_IB = _B // _GRID


def _tc_project(embt_ref, w2_ref, w1_ref, b1_ref, b2_ref,
                d_ref, gc_ref):
    w2 = w2_ref[...]
    dwr = w2[1:2, :] - w2[0:1, :]
    blk = embt_ref[...]
    s = jnp.dot(dwr, blk, preferred_element_type=jnp.float32)
    d_ref[...] = s.reshape(_CB // 128, 128)

    @pl.when(pl.program_id(0) == 0)
    def _():
        w1 = w1_ref[...]
        b1 = b1_ref[...]
        b2 = b2_ref[...]
        cd = b1[0, 0] * jnp.sum(dwr) + (b2[0, 1] - b2[0, 0])
        gc_ref[0:1, :] = jnp.full((1, 128), w1[0, 0], jnp.float32)
        gc_ref[1:2, :] = jnp.full((1, 128), w1[0, 1], jnp.float32)
        gc_ref[2:3, :] = jnp.full((1, 128), cd, jnp.float32)


def _sc_body(idx_hbm, d16_hbm, g_hbm, out_hbm, idx_v, hi_v, rows_v, g_v, out_v, sem):
    wid = lax.axis_index("s") * _NC + lax.axis_index("c")

    pltpu.sync_copy(idx_hbm.at[wid], idx_v)
    pltpu.sync_copy(g_hbm, g_v)

    for k in range(_KCH):
        for t in range(_CW // 16):
            v = idx_v[k, pl.ds(t * 16, 16)]
            hi_v[k, pl.ds(t * 16, 16)] = lax.shift_right_logical(v, 4)

    copies = [
        pltpu.async_copy(
            d16_hbm.at[hi_v.at[k]], rows_v.at[pl.ds(k * _CW, _CW)], sem
        )
        for k in range(_KCH)
    ]
    for c in copies:
        c.wait()

    iot = lax.iota(jnp.int32, 16)
    zero16 = jnp.zeros((16,), jnp.int32)
    one16 = jnp.ones((16,), jnp.int32)
    w0v = g_v[0, pl.ds(0, 16)]
    w1v = g_v[1, pl.ds(0, 16)]
    cdv = g_v[2, pl.ds(0, 16)]

    def group(g, carry):
        r0 = g * 32 + iot * 2
        r1 = r0 + 1
        lo0 = plsc.load_gather(idx_v, [lax.shift_right_logical(r0, 7), r0 & 127]) & 15
        lo1 = plsc.load_gather(idx_v, [lax.shift_right_logical(r1, 7), r1 & 127]) & 15
        v0 = plsc.load_gather(rows_v, [r0, lo0])
        v1 = plsc.load_gather(rows_v, [r1, lo1])
        d = w0v * v0 + w1v * v1 + cdv
        e = jnp.exp(d)
        o0 = 1.0 / (1.0 + e)
        o1 = 1.0 - o0
        bi = g * 16 + iot
        plsc.store_scatter(out_v, [bi, zero16], o0)
        plsc.store_scatter(out_v, [bi, one16], o1)
        return carry

    lax.fori_loop(0, _NG, group, 0)

    pltpu.sync_copy(out_v, out_hbm.at[pl.ds(wid * _BPW, _BPW)])


def kernel(input, emb, W1, b1, W2, b2):
    idx = input.astype(jnp.int32).reshape(_NW, _KCH, _CW)

    d2, gc = pl.pallas_call(
        _tc_project,
        grid=(_GRID,),
        in_specs=[
            pl.BlockSpec((_H, _CB), lambda i: (0, i)),
            pl.BlockSpec((2, _H), lambda i: (0, 0)),
            pl.BlockSpec((1, 2), lambda i: (0, 0)),
            pl.BlockSpec((1, 1), lambda i: (0, 0)),
            pl.BlockSpec((1, 2), lambda i: (0, 0)),
        ],
        out_specs=[
            pl.BlockSpec((_CB // 128, 128), lambda i: (i, 0)),
            pl.BlockSpec((3, 128), lambda i: (0, 0)),
        ],
        out_shape=[
            jax.ShapeDtypeStruct((_DROWS, 128), jnp.float32),
            jax.ShapeDtypeStruct((3, 128), jnp.float32),
        ],
    )(emb.T, W2, W1, b1.reshape(1, 1), b2.reshape(1, 2))
    d16 = d2.reshape(_DROWS * 8, _H)

    mesh = plsc.VectorSubcoreMesh(
        core_axis_name="c", subcore_axis_name="s", num_cores=_NC, num_subcores=_NS
    )
    run = pl.kernel(
        _sc_body,
        out_type=jax.ShapeDtypeStruct((_B, 2), jnp.float32),
        mesh=mesh,
        compiler_params=pltpu.CompilerParams(
            needs_layout_passes=False, use_tc_tiling_on_sc=False
        ),
        scratch_types=[
            pltpu.VMEM((_KCH, _CW), jnp.int32),
            pltpu.VMEM((_KCH, _CW), jnp.int32),
            pltpu.VMEM((_RPW, _H), jnp.float32),
            pltpu.VMEM((3, 128), jnp.float32),
            pltpu.VMEM((_BPW, 2), jnp.float32),
            pltpu.SemaphoreType.DMA,
        ],
    )
    return run(idx, d16, gc)


# speedup vs baseline: 1.7739x; 1.5853x over previous
import jax
import jax.numpy as jnp
from jax import lax
from jax.experimental import pallas as pl
from jax.experimental.pallas import tpu as pltpu
from jax.experimental.pallas import tpu_sc as plsc

_V = 1000000
_H = 16
_B = 16384

_NC = 2
_NS = 16
_NW = _NC * _NS
_BPW = _B // _NW
_RPW = 2 * _BPW
_KCH = 8
_CW = _RPW // _KCH
_NG = _BPW // 16
_NJ = _B // 128
_JPW = _NJ // _NW

_CB = 131072
_GRID = -(-_V // _CB)
_DROWS = _GRID * _CB // 128


def _tc_project(embt_ref, w2_ref, w1_ref, b1_ref, b2_ref, d_ref, gc_ref):
    w2 = w2_ref[...]
    dwr = w2[1:2, :] - w2[0:1, :]
    blk = embt_ref[...]
    s = jnp.dot(dwr, blk, preferred_element_type=jnp.float32)
    d_ref[...] = s.reshape(_CB // 128, 128)

    @pl.when(pl.program_id(0) == 0)
    def _():
        w1 = w1_ref[...]
        b1 = b1_ref[...]
        b2 = b2_ref[...]
        cd = b1[0, 0] * jnp.sum(dwr) + (b2[0, 1] - b2[0, 0])
        gc_ref[0:1, :] = jnp.full((1, 128), w1[0, 0], jnp.float32)
        gc_ref[1:2, :] = jnp.full((1, 128), w1[0, 1], jnp.float32)
        gc_ref[2:3, :] = jnp.full((1, 128), cd, jnp.float32)


def _sc_body(idx_hbm, d16_hbm, g_hbm, out_hbm, idx_v, hi_v, rows_v, g_v, out_v, sem):
    wid = lax.axis_index("s") * _NC + lax.axis_index("c")

    pltpu.sync_copy(idx_hbm.at[pl.ds(wid * _JPW, _JPW)], idx_v)
    pltpu.sync_copy(g_hbm, g_v)

    for j in range(_JPW):
        for c in range(2):
            for t in range(_CW // 16):
                v = idx_v[j, c, pl.ds(t * 16, 16)]
                hi_v[2 * j + c, pl.ds(t * 16, 16)] = lax.shift_right_logical(v, 4)

    copies = [
        pltpu.async_copy(
            d16_hbm.at[hi_v.at[k]], rows_v.at[pl.ds(k * _CW, _CW)], sem
        )
        for k in range(_KCH)
    ]
    for c in copies:
        c.wait()

    iot = lax.iota(jnp.int32, 16)
    zero16 = jnp.zeros((16,), jnp.int32)
    one16 = jnp.ones((16,), jnp.int32)
    w0v = g_v[0, pl.ds(0, 16)]
    w1v = g_v[1, pl.ds(0, 16)]
    cdv = g_v[2, pl.ds(0, 16)]

    def group(g, carry):
        b = g * 16 + iot
        jv = lax.shift_right_logical(b, 7)
        lv = b & 127
        lo0 = plsc.load_gather(idx_v, [jv, zero16, lv]) & 15
        lo1 = plsc.load_gather(idx_v, [jv, one16, lv]) & 15
        r0 = jv * 256 + lv
        r1 = r0 + 128
        v0 = plsc.load_gather(rows_v, [r0, lo0])
        v1 = plsc.load_gather(rows_v, [r1, lo1])
        d = w0v * v0 + w1v * v1 + cdv
        e = jnp.exp(d)
        o0 = 1.0 / (1.0 + e)
        o1 = 1.0 - o0
        plsc.store_scatter(out_v, [jv, zero16, lv], o0)
        plsc.store_scatter(out_v, [jv, one16, lv], o1)
        return carry

    lax.fori_loop(0, _NG, group, 0)

    pltpu.sync_copy(out_v, out_hbm.at[pl.ds(wid * _JPW, _JPW)])


def kernel(input, emb, W1, b1, W2, b2):
    idx3 = jnp.swapaxes(input.astype(jnp.int32).T.reshape(2, _NJ, 128), 0, 1)

    d2, gc = pl.pallas_call(
        _tc_project,
        grid=(_GRID,),
        in_specs=[
            pl.BlockSpec((_H, _CB), lambda i: (0, i)),
            pl.BlockSpec((2, _H), lambda i: (0, 0)),
            pl.BlockSpec((1, 2), lambda i: (0, 0)),
            pl.BlockSpec((1, 1), lambda i: (0, 0)),
            pl.BlockSpec((1, 2), lambda i: (0, 0)),
        ],
        out_specs=[
            pl.BlockSpec((_CB // 128, 128), lambda i: (i, 0)),
            pl.BlockSpec((3, 128), lambda i: (0, 0)),
        ],
        out_shape=[
            jax.ShapeDtypeStruct((_DROWS, 128), jnp.float32),
            jax.ShapeDtypeStruct((3, 128), jnp.float32),
        ],
    )(emb.T, W2, W1, b1.reshape(1, 1), b2.reshape(1, 2))
    d16 = d2.reshape(_DROWS * 8, _H)

    mesh = plsc.VectorSubcoreMesh(
        core_axis_name="c", subcore_axis_name="s", num_cores=_NC, num_subcores=_NS
    )
    run = pl.kernel(
        _sc_body,
        out_type=jax.ShapeDtypeStruct((_NJ, 2, 128), jnp.float32),
        mesh=mesh,
        compiler_params=pltpu.CompilerParams(
            needs_layout_passes=False, use_tc_tiling_on_sc=False
        ),
        scratch_types=[
            pltpu.VMEM((_JPW, 2, 128), jnp.int32),
            pltpu.VMEM((_KCH, _CW), jnp.int32),
            pltpu.VMEM((_RPW, _H), jnp.float32),
            pltpu.VMEM((3, 128), jnp.float32),
            pltpu.VMEM((_JPW, 2, 128), jnp.float32),
            pltpu.SemaphoreType.DMA,
        ],
    )
    out3 = run(idx3, d16, gc)
    return jnp.swapaxes(out3, 0, 1).reshape(2, _B).T
